# Initial kernel scaffold; baseline (speedup 1.0000x reference)
import jax
import jax.numpy as jnp
from jax.experimental import pallas as pl


def kernel(policy, value_predictions, actions):
    raise NotImplementedError("write your pallas kernel here")


# speedup vs baseline: 2.0326x; 2.0326x over previous
import functools

import jax
import jax.numpy as jnp
from jax.experimental import pallas as pl

_C = 1000


def _fused_kernel(p_ref, a_ref, alp_ref, ent_ref):
    x = p_ref[...]
    a = a_ref[...]
    m = jnp.max(x, axis=1, keepdims=True)
    xm = x - m
    e = jnp.exp(xm)
    s = jnp.sum(e, axis=1, keepdims=True)
    t = jnp.sum(xm * e, axis=1, keepdims=True)
    logs = jnp.log(s)
    col = jax.lax.broadcasted_iota(jnp.int32, x.shape, 1)
    sel = jnp.sum(jnp.where(col == a, xm, 0.0), axis=1, keepdims=True)
    alp_ref[...] = sel - logs
    block_ent = jnp.sum(logs - t / s).reshape(1, 1)
    i = pl.program_id(0)
    prev = jnp.where(i == 0, jnp.zeros((1, 1), jnp.float32), ent_ref[...])
    ent_ref[...] = prev + block_ent


@functools.partial(jax.jit, static_argnames=("rows_per_block",))
def _run(policy_flat, actions_flat, rows_per_block=512):
    n, c = policy_flat.shape
    grid = n // rows_per_block
    alp, ent = pl.pallas_call(
        _fused_kernel,
        grid=(grid,),
        in_specs=[
            pl.BlockSpec((rows_per_block, c), lambda i: (i, 0)),
            pl.BlockSpec((rows_per_block, 1), lambda i: (i, 0)),
        ],
        out_specs=[
            pl.BlockSpec((rows_per_block, 1), lambda i: (i, 0)),
            pl.BlockSpec((1, 1), lambda i: (0, 0)),
        ],
        out_shape=[
            jax.ShapeDtypeStruct((n, 1), jnp.float32),
            jax.ShapeDtypeStruct((1, 1), jnp.float32),
        ],
    )(policy_flat, actions_flat)
    return alp, ent


def kernel(policy, value_predictions, actions):
    b = policy.shape[0]
    flat = policy.reshape(-1, _C)
    a_flat = actions.reshape(-1, 1).astype(jnp.int32)
    alp, ent = _run(flat, a_flat)
    action_log_probs = alp.reshape(b, -1)
    dist_entropy = (ent[0, 0] / b).astype(jnp.float32)
    return (value_predictions, action_log_probs, dist_entropy)


# speedup vs baseline: 2.1544x; 1.0599x over previous
import functools

import jax
import jax.numpy as jnp
from jax.experimental import pallas as pl

_C = 1000


def _fused_kernel(p_ref, a_ref, alp_ref, ent_ref):
    x = p_ref[...]
    a = a_ref[...]
    m = jnp.max(x, axis=1, keepdims=True)
    xm = x - m
    e = jnp.exp(xm)
    s = jnp.sum(e, axis=1, keepdims=True)
    t = jnp.sum(xm * e, axis=1, keepdims=True)
    logs = jnp.log(s)
    col = jax.lax.broadcasted_iota(jnp.int32, x.shape, 1)
    sel = jnp.sum(jnp.where(col == a, xm, 0.0), axis=1, keepdims=True)
    alp_ref[...] = sel - logs
    block_ent = jnp.sum(logs - t / s).reshape(1, 1)
    i = pl.program_id(0)
    prev = jnp.where(i == 0, jnp.zeros((1, 1), jnp.float32), ent_ref[...])
    ent_ref[...] = prev + block_ent


@functools.partial(jax.jit, static_argnames=("rows_per_block",))
def _run(policy_flat, actions_flat, rows_per_block=1024):
    n, c = policy_flat.shape
    grid = n // rows_per_block
    alp, ent = pl.pallas_call(
        _fused_kernel,
        grid=(grid,),
        in_specs=[
            pl.BlockSpec((rows_per_block, c), lambda i: (i, 0)),
            pl.BlockSpec((rows_per_block, 1), lambda i: (i, 0)),
        ],
        out_specs=[
            pl.BlockSpec((rows_per_block, 1), lambda i: (i, 0)),
            pl.BlockSpec((1, 1), lambda i: (0, 0)),
        ],
        out_shape=[
            jax.ShapeDtypeStruct((n, 1), jnp.float32),
            jax.ShapeDtypeStruct((1, 1), jnp.float32),
        ],
    )(policy_flat, actions_flat)
    return alp, ent


def kernel(policy, value_predictions, actions):
    b = policy.shape[0]
    flat = policy.reshape(-1, _C)
    a_flat = actions.reshape(-1, 1).astype(jnp.int32)
    alp, ent = _run(flat, a_flat)
    action_log_probs = alp.reshape(b, -1)
    dist_entropy = (ent[0, 0] / b).astype(jnp.float32)
    return (value_predictions, action_log_probs, dist_entropy)


# speedup vs baseline: 2.2086x; 1.0251x over previous
import functools

import jax
import jax.numpy as jnp
from jax.experimental import pallas as pl

_C = 1000


def _fused_kernel(p_ref, a_ref, alp_ref, ent_ref):
    x = p_ref[...]
    a = a_ref[...]
    m = jnp.max(x, axis=1, keepdims=True)
    xm = x - m
    e = jnp.exp(xm)
    s = jnp.sum(e, axis=1, keepdims=True)
    t = jnp.sum(xm * e, axis=1, keepdims=True)
    logs = jnp.log(s)
    col = jax.lax.broadcasted_iota(jnp.int32, x.shape, 1)
    sel = jnp.sum(jnp.where(col == a, xm, 0.0), axis=1, keepdims=True)
    alp_ref[...] = sel - logs
    block_ent = jnp.sum(logs - t / s).reshape(1, 1)
    i = pl.program_id(0)
    prev = jnp.where(i == 0, jnp.zeros((1, 1), jnp.float32), ent_ref[...])
    ent_ref[...] = prev + block_ent


@functools.partial(jax.jit, static_argnames=("rows_per_block",))
def _run(policy_flat, actions_flat, rows_per_block=2048):
    n, c = policy_flat.shape
    grid = n // rows_per_block
    alp, ent = pl.pallas_call(
        _fused_kernel,
        grid=(grid,),
        in_specs=[
            pl.BlockSpec((rows_per_block, c), lambda i: (i, 0)),
            pl.BlockSpec((rows_per_block, 1), lambda i: (i, 0)),
        ],
        out_specs=[
            pl.BlockSpec((rows_per_block, 1), lambda i: (i, 0)),
            pl.BlockSpec((1, 1), lambda i: (0, 0)),
        ],
        out_shape=[
            jax.ShapeDtypeStruct((n, 1), jnp.float32),
            jax.ShapeDtypeStruct((1, 1), jnp.float32),
        ],
    )(policy_flat, actions_flat)
    return alp, ent


def kernel(policy, value_predictions, actions):
    b = policy.shape[0]
    flat = policy.reshape(-1, _C)
    a_flat = actions.reshape(-1, 1).astype(jnp.int32)
    alp, ent = _run(flat, a_flat)
    action_log_probs = alp.reshape(b, -1)
    dist_entropy = (ent[0, 0] / b).astype(jnp.float32)
    return (value_predictions, action_log_probs, dist_entropy)
